# baseline (speedup 1.0000x reference)
import functools

import jax
import jax.numpy as jnp
from jax import lax
from jax.experimental import pallas as pl
from jax.experimental.pallas import tpu as pltpu
from jax.experimental.pallas import tpu_sc as plsc

N = 10000
E = 320000
EP = 327680
NROWS = EP // 128
NP = 10240
NW = 32
RPW = NROWS // NW
KJ = 10
NMAC = RPW // KJ
RSUB = NP // 16
B = 100
NPG = 100
KP = 32

_mesh = plsc.VectorSubcoreMesh(core_axis_name="c", subcore_axis_name="s")


def _fill(buf, rows, width, value):
    vec = jnp.full((16,), value, jnp.float32)

    def body(i, carry):
        for j in range(width // 16):
            buf[i, pl.ds(j * 16, 16)] = vec
        return carry

    lax.fori_loop(0, rows, body, 0)


def _sc_partials(feat, with_gather):
    nrow_banks = 2 if with_gather else 1
    scratch = [
        pltpu.VMEM((3, KJ, 128), jnp.int32),
        pltpu.VMEM((nrow_banks, KJ, 128, feat), jnp.float32),
        pltpu.VMEM_SHARED((NP, feat), jnp.float32),
        pltpu.SemaphoreType.DMA,
        pltpu.SemaphoreType.DMA,
        pltpu.SemaphoreType.DMA,
        pltpu.SemaphoreType.DMA,
    ]
    if with_gather:
        scratch.insert(0, pltpu.VMEM((3, KJ, 128), jnp.int32))
        scratch.append(pltpu.VMEM_SHARED((N, feat), jnp.float32))

    @functools.partial(
        pl.kernel,
        out_type=jax.ShapeDtypeStruct((2, N, feat), jnp.float32),
        mesh=_mesh,
        scratch_types=scratch,
        compiler_params=pltpu.CompilerParams(use_tc_tiling_on_sc=False, skip_device_barrier=True),
    )
    def run(*refs):
        if with_gather:
            (zt_hbm, srcr_hbm, dstr_hbm, out_hbm,
             sidx, didx, rows, acc, gsem, isem, ssem0, ssem1, table) = refs
        else:
            (dstr_hbm, out_hbm, didx, rows, acc,
             gsem, isem, ssem0, ssem1) = refs
        ssems = (ssem0, ssem1)
        cid = lax.axis_index("c")
        sid = lax.axis_index("s")
        wid = sid * 2 + cid
        if with_gather:
            pltpu.sync_copy(zt_hbm.at[pl.ds(sid * (N // 16), N // 16)],
                            table.at[pl.ds(sid * (N // 16), N // 16)])
        zb = rows.at[0, 0]
        _fill(zb, 128, feat, 0.0)
        for t in range(RSUB // 128):
            pltpu.sync_copy(zb, acc.at[pl.ds(sid * RSUB + t * 128, 128)])
        if not with_gather:
            for j in range(KJ):
                _fill(rows.at[0, j], 128, feat, 1.0)
        plsc.subcore_barrier()

        row0 = wid * RPW

        def fetch_idx(m):
            r = row0 + m * KJ
            b3 = m % 3
            ds = [pltpu.async_copy(dstr_hbm.at[pl.ds(r, KJ)],
                                   didx.at[b3], isem)]
            if with_gather:
                ds.append(pltpu.async_copy(srcr_hbm.at[pl.ds(r, KJ)],
                                           sidx.at[b3], isem))
            return ds

        def fire_gathers(m):
            return [pltpu.async_copy(table.at[sidx.at[m % 3, j]],
                                     rows.at[m % 2, j], gsem)
                    for j in range(KJ)]

        def fire_scatters(m):
            rb = (m % 2) if with_gather else 0
            return [pltpu.async_copy(rows.at[rb, j],
                                     acc.at[didx.at[m % 3, j]],
                                     ssems[m % 2], add=True)
                    for j in range(KJ)]

        idx_d = fetch_idx(0)
        for d in idx_d:
            d.wait()
        gd = fire_gathers(0) if with_gather else None
        idx_next = fetch_idx(1) if NMAC > 1 else []
        sd_prev = []
        for m in range(NMAC):
            if with_gather:
                for d in gd:
                    d.wait()
            for d in sd_prev:
                d.wait()
            if m + 1 < NMAC:
                for d in idx_next:
                    d.wait()
                if with_gather:
                    gd = fire_gathers(m + 1)
            sd_prev = fire_scatters(m)
            if m + 2 < NMAC:
                idx_next = fetch_idx(m + 2)
        for d in sd_prev:
            d.wait()
        plsc.subcore_barrier()
        pltpu.sync_copy(acc.at[pl.ds(sid * (N // 16), N // 16)],
                        out_hbm.at[cid, pl.ds(sid * (N // 16), N // 16)])

    return run


_sc_spmm32 = _sc_partials(32, True)
_sc_spmm16 = _sc_partials(16, True)
_sc_deg = _sc_partials(16, False)


@functools.partial(
    pl.kernel,
    out_type=jax.ShapeDtypeStruct((NW, NPG, 128), jnp.float32),
    mesh=_mesh,
    scratch_types=[
        pltpu.VMEM((NPG,), jnp.int32),
        pltpu.VMEM((NPG, 128), jnp.float32),
        pltpu.SemaphoreType.DMA,
    ],
    compiler_params=pltpu.CompilerParams(use_tc_tiling_on_sc=False, skip_device_barrier=True),
)
def _sc_gather(cat_hbm, sel_hbm, out_hbm, idx, rows, sem):
    cid = lax.axis_index("c")
    sid = lax.axis_index("s")
    wid = sid * 2 + cid
    pltpu.sync_copy(sel_hbm.at[wid], idx)
    pltpu.async_copy(cat_hbm.at[idx], rows, sem).wait()
    pltpu.sync_copy(rows, out_hbm.at[wid])


def _pc(body, out_shape):
    return pl.pallas_call(body, out_shape=out_shape)


NB = 10
BL = N // NB


def _rows(feat):
    return pl.BlockSpec((BL, feat), lambda g: (g, 0))


def _part(feat):
    return pl.BlockSpec((2, BL, feat), lambda g: (0, g, 0))


def _full(*shape):
    return pl.BlockSpec(shape, lambda g: tuple(0 for _ in shape))


def _pcg(body, in_specs, out_specs, out_shape):
    return pl.pallas_call(body, grid=(NB,), in_specs=in_specs,
                          out_specs=out_specs, out_shape=out_shape)


def _mm_body(x_ref, w_ref, o_ref):
    o_ref[...] = jnp.dot(x_ref[...], w_ref[...],
                         preferred_element_type=jnp.float32)


def _prep_body(degp_ref, xw_ref, dinv_ref, zt_ref):
    deg = degp_ref[0, :, 0:1] + degp_ref[1, :, 0:1] + 1.0
    dinv = lax.rsqrt(jnp.maximum(deg, 1.0))
    dinv_ref[...] = dinv
    zt_ref[...] = dinv * xw_ref[...]


def _layer_body(s_ref, zt_ref, dinv_ref, b_ref, w_ref, h_ref, ztn_ref):
    s = s_ref[0] + s_ref[1] + zt_ref[...]
    dinv = dinv_ref[...]
    h = jnp.tanh(s * dinv + b_ref[...])
    h_ref[...] = h
    ztn_ref[...] = dinv * jnp.dot(h, w_ref[...],
                                  preferred_element_type=jnp.float32)


def _h3cat_body(s_ref, zt_ref, dinv_ref, b_ref, h0_ref, h1_ref, h2_ref,
                cat_ref):
    s = s_ref[0] + s_ref[1] + zt_ref[...]
    h3 = jnp.tanh(s * dinv_ref[...] + b_ref[...])[:, 0:1]
    cat_ref[...] = jnp.concatenate(
        [h0_ref[...], h1_ref[...], h2_ref[...], h3,
         jnp.zeros((BL, 31), jnp.float32)], axis=1)


def _sel_body(keys_ref, sel_ref):
    k0 = keys_ref[...]
    ion = lax.broadcasted_iota(jnp.int32, (B, NPG), 1)
    iok = lax.broadcasted_iota(jnp.int32, (B, KP), 1)

    def step(t, carry):
        taken, sel = carry
        kk = jnp.where(taken != 0, -2.0, k0)
        m = jnp.max(kk, axis=1, keepdims=True)
        nidx = jnp.min(jnp.where(kk == m, ion, NPG), axis=1, keepdims=True)
        sel = sel + jnp.where(iok == t, nidx, 0)
        taken = taken | (ion == nidx).astype(jnp.int32)
        return taken, sel

    _, sel = lax.fori_loop(
        0, 30, step,
        (jnp.zeros((B, NPG), jnp.int32), jnp.zeros((B, KP), jnp.int32)))
    sel_ref[...] = sel + NPG * lax.broadcasted_iota(jnp.int32, (B, KP), 0)


def _head_body(xs_ref, c1w_ref, c1b_ref, w2_ref, c2b_ref, l1_ref, l1b_ref,
               l2_ref, l2b_ref, out_ref):
    y1 = jnp.maximum(
        jnp.dot(xs_ref[...], c1w_ref[...],
                preferred_element_type=jnp.float32) + c1b_ref[...], 0.0)
    pooled = jnp.max(y1.reshape(B * 16, 2, 16), axis=1)
    p3 = pooled.reshape(B, 16, 16)
    patches = jnp.concatenate([p3[:, k:k + 11, :] for k in range(5)], axis=2)
    y2 = jnp.maximum(
        jnp.dot(patches.reshape(B * 11, 80), w2_ref[...],
                preferred_element_type=jnp.float32) + c2b_ref[...], 0.0)
    y23 = y2.reshape(B, 11, 32)
    y3pre = jnp.zeros((B, 128), jnp.float32)
    for t in range(11):
        y3pre = y3pre + jnp.dot(y23[:, t, :], l1_ref[t * 32:(t + 1) * 32, :],
                                preferred_element_type=jnp.float32)
    y3 = jnp.maximum(y3pre + l1b_ref[...], 0.0)
    logits = jnp.dot(y3, l2_ref[...],
                     preferred_element_type=jnp.float32) + l2b_ref[...]
    m = jnp.max(logits, axis=1, keepdims=True)
    z = logits - m
    out_ref[...] = z - jnp.log(jnp.sum(jnp.exp(z), axis=1, keepdims=True))


def kernel(x, edge_index, batch, gcn_w0, gcn_b0, gcn_w1, gcn_b1, gcn_w2,
           gcn_b2, gcn_w3, gcn_b3, c1_w, c1_b, c2_w, c2_b, l1_w, l1_b,
           l2_w, l2_b):
    del batch
    f32 = jnp.float32
    src = edge_index[0]
    dst = edge_index[1]
    srcr = jnp.concatenate(
        [src, jnp.zeros((EP - E,), jnp.int32)]).reshape(NROWS, 128)
    dstr = jnp.concatenate(
        [dst, jnp.full((EP - E,), N, jnp.int32)]).reshape(NROWS, 128)

    degp = _sc_deg(dstr)
    xw0 = _pcg(_mm_body, [_rows(128), _full(128, 32)], _rows(32),
               jax.ShapeDtypeStruct((N, 32), f32))(x, gcn_w0)
    dinv, zt0 = _pcg(
        _prep_body, [_part(16), _rows(32)], (_rows(1), _rows(32)),
        (jax.ShapeDtypeStruct((N, 1), f32),
         jax.ShapeDtypeStruct((N, 32), f32)))(degp, xw0)

    def layer(fnext, s, zt, b, w):
        return _pcg(
            _layer_body,
            [_part(32), _rows(32), _rows(1), _full(1, 32), _full(32, fnext)],
            (_rows(32), _rows(fnext)),
            (jax.ShapeDtypeStruct((N, 32), f32),
             jax.ShapeDtypeStruct((N, fnext), f32)))(
            s, zt, dinv, b.reshape(1, 32), w)

    s0 = _sc_spmm32(zt0, srcr, dstr)
    h0, zt1 = layer(32, s0, zt0, gcn_b0, gcn_w1)
    s1 = _sc_spmm32(zt1, srcr, dstr)
    h1, zt2 = layer(32, s1, zt1, gcn_b1, gcn_w2)
    s2 = _sc_spmm32(zt2, srcr, dstr)
    w3p = jnp.pad(gcn_w3, ((0, 0), (0, 15)))
    h2, zt3 = layer(16, s2, zt2, gcn_b2, w3p)
    s3 = _sc_spmm16(zt3, srcr, dstr)
    cat = _pcg(
        _h3cat_body,
        [_part(16), _rows(16), _rows(1), _full(1, 1),
         _rows(32), _rows(32), _rows(32)],
        _rows(128), jax.ShapeDtypeStruct((N, 128), f32))(
        s3, zt3, dinv, gcn_b3.reshape(1, 1), h0, h1, h2)

    keys = lax.slice(cat, (0, 96), (N, 97)).reshape(B, NPG)
    selg = _pc(_sel_body, jax.ShapeDtypeStruct((B, KP), jnp.int32))(keys)
    sel2 = selg.reshape(NW, NPG)
    xs_all = _sc_gather(cat, sel2).reshape(B * KP, 128)

    c1wr = jnp.pad(c1_w.reshape(16, 97).T, ((0, 31), (0, 0)))
    w2r = c2_w.transpose(2, 1, 0).reshape(80, 32)
    l1wr = l1_w.reshape(32, 11, 128).transpose(1, 0, 2).reshape(352, 128)
    out = _pc(_head_body, jax.ShapeDtypeStruct((B, 10), f32))(
        xs_all, c1wr, c1_b.reshape(1, 16), w2r, c2_b.reshape(1, 32),
        l1wr, l1_b.reshape(1, 128), l2_w, l2_b.reshape(1, 10))
    return out


# speedup vs baseline: 1.0080x; 1.0080x over previous
import functools

import jax
import jax.numpy as jnp
from jax import lax
from jax.experimental import pallas as pl
from jax.experimental.pallas import tpu as pltpu
from jax.experimental.pallas import tpu_sc as plsc

N = 10000
E = 320000
EP = 327680
NROWS = EP // 128
NP = 10240
NW = 32
RPW = NROWS // NW
KJ = 8
NMAC = RPW // KJ
RSUB = NP // 16
B = 100
NPG = 100
KP = 32

_mesh = plsc.VectorSubcoreMesh(core_axis_name="c", subcore_axis_name="s")


def _fill(buf, rows, width, value):
    vec = jnp.full((16,), value, jnp.float32)

    def body(i, carry):
        for j in range(width // 16):
            buf[i, pl.ds(j * 16, 16)] = vec
        return carry

    lax.fori_loop(0, rows, body, 0)


def _sc_partials(feat, with_gather):
    nrow_banks = 2 if with_gather else 1
    scratch = [
        pltpu.VMEM((3, KJ, 128), jnp.int32),
        pltpu.VMEM((nrow_banks, KJ, 128, feat), jnp.float32),
        pltpu.VMEM_SHARED((NP, feat), jnp.float32),
        pltpu.SemaphoreType.DMA,
        pltpu.SemaphoreType.DMA,
        pltpu.SemaphoreType.DMA,
        pltpu.SemaphoreType.DMA,
    ]
    if with_gather:
        scratch.insert(0, pltpu.VMEM((3, KJ, 128), jnp.int32))
        scratch.append(pltpu.VMEM_SHARED((N, feat), jnp.float32))

    @functools.partial(
        pl.kernel,
        out_type=jax.ShapeDtypeStruct((2, N, feat), jnp.float32),
        mesh=_mesh,
        scratch_types=scratch,
        compiler_params=pltpu.CompilerParams(use_tc_tiling_on_sc=False, skip_device_barrier=True),
    )
    def run(*refs):
        if with_gather:
            (zt_hbm, srcr_hbm, dstr_hbm, out_hbm,
             sidx, didx, rows, acc, gsem, isem, ssem0, ssem1, table) = refs
        else:
            (dstr_hbm, out_hbm, didx, rows, acc,
             gsem, isem, ssem0, ssem1) = refs
        ssems = (ssem0, ssem1)
        cid = lax.axis_index("c")
        sid = lax.axis_index("s")
        wid = sid * 2 + cid
        if with_gather:
            pltpu.sync_copy(zt_hbm.at[pl.ds(sid * (N // 16), N // 16)],
                            table.at[pl.ds(sid * (N // 16), N // 16)])
        zb = rows.at[0, 0]
        _fill(zb, 128, feat, 0.0)
        for t in range(RSUB // 128):
            pltpu.sync_copy(zb, acc.at[pl.ds(sid * RSUB + t * 128, 128)])
        if not with_gather:
            for j in range(KJ):
                _fill(rows.at[0, j], 128, feat, 1.0)
        plsc.subcore_barrier()

        row0 = wid * RPW

        def fetch_idx(m):
            r = row0 + m * KJ
            b3 = m % 3
            ds = [pltpu.async_copy(dstr_hbm.at[pl.ds(r, KJ)],
                                   didx.at[b3], isem)]
            if with_gather:
                ds.append(pltpu.async_copy(srcr_hbm.at[pl.ds(r, KJ)],
                                           sidx.at[b3], isem))
            return ds

        def fire_gathers(m):
            return [pltpu.async_copy(table.at[sidx.at[m % 3, j]],
                                     rows.at[m % 2, j], gsem)
                    for j in range(KJ)]

        def fire_scatters(m):
            rb = (m % 2) if with_gather else 0
            return [pltpu.async_copy(rows.at[rb, j],
                                     acc.at[didx.at[m % 3, j]],
                                     ssems[m % 2], add=True)
                    for j in range(KJ)]

        idx_d = fetch_idx(0)
        for d in idx_d:
            d.wait()
        gd = fire_gathers(0) if with_gather else None
        idx_next = fetch_idx(1) if NMAC > 1 else []
        sd_prev = []
        for m in range(NMAC):
            if with_gather:
                for d in gd:
                    d.wait()
            for d in sd_prev:
                d.wait()
            if m + 1 < NMAC:
                for d in idx_next:
                    d.wait()
                if with_gather:
                    gd = fire_gathers(m + 1)
            sd_prev = fire_scatters(m)
            if m + 2 < NMAC:
                idx_next = fetch_idx(m + 2)
        for d in sd_prev:
            d.wait()
        plsc.subcore_barrier()
        pltpu.sync_copy(acc.at[pl.ds(sid * (N // 16), N // 16)],
                        out_hbm.at[cid, pl.ds(sid * (N // 16), N // 16)])

    return run


_sc_spmm32 = _sc_partials(32, True)
_sc_spmm16 = _sc_partials(16, True)
_sc_deg = _sc_partials(16, False)


@functools.partial(
    pl.kernel,
    out_type=jax.ShapeDtypeStruct((NW, NPG, 128), jnp.float32),
    mesh=_mesh,
    scratch_types=[
        pltpu.VMEM((NPG,), jnp.int32),
        pltpu.VMEM((NPG, 128), jnp.float32),
        pltpu.SemaphoreType.DMA,
    ],
    compiler_params=pltpu.CompilerParams(use_tc_tiling_on_sc=False, skip_device_barrier=True),
)
def _sc_gather(cat_hbm, sel_hbm, out_hbm, idx, rows, sem):
    cid = lax.axis_index("c")
    sid = lax.axis_index("s")
    wid = sid * 2 + cid
    pltpu.sync_copy(sel_hbm.at[wid], idx)
    pltpu.async_copy(cat_hbm.at[idx], rows, sem).wait()
    pltpu.sync_copy(rows, out_hbm.at[wid])


def _pc(body, out_shape):
    return pl.pallas_call(body, out_shape=out_shape)


NB = 10
BL = N // NB


def _rows(feat):
    return pl.BlockSpec((BL, feat), lambda g: (g, 0))


def _part(feat):
    return pl.BlockSpec((2, BL, feat), lambda g: (0, g, 0))


def _full(*shape):
    return pl.BlockSpec(shape, lambda g: tuple(0 for _ in shape))


def _pcg(body, in_specs, out_specs, out_shape):
    return pl.pallas_call(body, grid=(NB,), in_specs=in_specs,
                          out_specs=out_specs, out_shape=out_shape)


def _mm_body(x_ref, w_ref, o_ref):
    o_ref[...] = jnp.dot(x_ref[...], w_ref[...],
                         preferred_element_type=jnp.float32)


def _prep_body(degp_ref, xw_ref, dinv_ref, zt_ref):
    deg = degp_ref[0, :, 0:1] + degp_ref[1, :, 0:1] + 1.0
    dinv = lax.rsqrt(jnp.maximum(deg, 1.0))
    dinv_ref[...] = dinv
    zt_ref[...] = dinv * xw_ref[...]


def _layer_body(s_ref, zt_ref, dinv_ref, b_ref, w_ref, h_ref, ztn_ref):
    s = s_ref[0] + s_ref[1] + zt_ref[...]
    dinv = dinv_ref[...]
    h = jnp.tanh(s * dinv + b_ref[...])
    h_ref[...] = h
    ztn_ref[...] = dinv * jnp.dot(h, w_ref[...],
                                  preferred_element_type=jnp.float32)


def _h3cat_body(s_ref, zt_ref, dinv_ref, b_ref, h0_ref, h1_ref, h2_ref,
                cat_ref):
    s = s_ref[0] + s_ref[1] + zt_ref[...]
    h3 = jnp.tanh(s * dinv_ref[...] + b_ref[...])[:, 0:1]
    cat_ref[...] = jnp.concatenate(
        [h0_ref[...], h1_ref[...], h2_ref[...], h3,
         jnp.zeros((BL, 31), jnp.float32)], axis=1)


def _sel_body(keys_ref, sel_ref):
    k0 = keys_ref[...]
    ion = lax.broadcasted_iota(jnp.int32, (B, NPG), 1)
    iok = lax.broadcasted_iota(jnp.int32, (B, KP), 1)

    def step(t, carry):
        taken, sel = carry
        kk = jnp.where(taken != 0, -2.0, k0)
        m = jnp.max(kk, axis=1, keepdims=True)
        nidx = jnp.min(jnp.where(kk == m, ion, NPG), axis=1, keepdims=True)
        sel = sel + jnp.where(iok == t, nidx, 0)
        taken = taken | (ion == nidx).astype(jnp.int32)
        return taken, sel

    _, sel = lax.fori_loop(
        0, 30, step,
        (jnp.zeros((B, NPG), jnp.int32), jnp.zeros((B, KP), jnp.int32)))
    sel_ref[...] = sel + NPG * lax.broadcasted_iota(jnp.int32, (B, KP), 0)


def _head_body(xs_ref, c1w_ref, c1b_ref, w2_ref, c2b_ref, l1_ref, l1b_ref,
               l2_ref, l2b_ref, out_ref):
    y1 = jnp.maximum(
        jnp.dot(xs_ref[...], c1w_ref[...],
                preferred_element_type=jnp.float32) + c1b_ref[...], 0.0)
    pooled = jnp.max(y1.reshape(B * 16, 2, 16), axis=1)
    p3 = pooled.reshape(B, 16, 16)
    patches = jnp.concatenate([p3[:, k:k + 11, :] for k in range(5)], axis=2)
    y2 = jnp.maximum(
        jnp.dot(patches.reshape(B * 11, 80), w2_ref[...],
                preferred_element_type=jnp.float32) + c2b_ref[...], 0.0)
    y23 = y2.reshape(B, 11, 32)
    y3pre = jnp.zeros((B, 128), jnp.float32)
    for t in range(11):
        y3pre = y3pre + jnp.dot(y23[:, t, :], l1_ref[t * 32:(t + 1) * 32, :],
                                preferred_element_type=jnp.float32)
    y3 = jnp.maximum(y3pre + l1b_ref[...], 0.0)
    logits = jnp.dot(y3, l2_ref[...],
                     preferred_element_type=jnp.float32) + l2b_ref[...]
    m = jnp.max(logits, axis=1, keepdims=True)
    z = logits - m
    out_ref[...] = z - jnp.log(jnp.sum(jnp.exp(z), axis=1, keepdims=True))


def kernel(x, edge_index, batch, gcn_w0, gcn_b0, gcn_w1, gcn_b1, gcn_w2,
           gcn_b2, gcn_w3, gcn_b3, c1_w, c1_b, c2_w, c2_b, l1_w, l1_b,
           l2_w, l2_b):
    del batch
    f32 = jnp.float32
    src = edge_index[0]
    dst = edge_index[1]
    srcr = jnp.concatenate(
        [src, jnp.zeros((EP - E,), jnp.int32)]).reshape(NROWS, 128)
    dstr = jnp.concatenate(
        [dst, jnp.full((EP - E,), N, jnp.int32)]).reshape(NROWS, 128)

    degp = _sc_deg(dstr)
    xw0 = _pcg(_mm_body, [_rows(128), _full(128, 32)], _rows(32),
               jax.ShapeDtypeStruct((N, 32), f32))(x, gcn_w0)
    dinv, zt0 = _pcg(
        _prep_body, [_part(16), _rows(32)], (_rows(1), _rows(32)),
        (jax.ShapeDtypeStruct((N, 1), f32),
         jax.ShapeDtypeStruct((N, 32), f32)))(degp, xw0)

    def layer(fnext, s, zt, b, w):
        return _pcg(
            _layer_body,
            [_part(32), _rows(32), _rows(1), _full(1, 32), _full(32, fnext)],
            (_rows(32), _rows(fnext)),
            (jax.ShapeDtypeStruct((N, 32), f32),
             jax.ShapeDtypeStruct((N, fnext), f32)))(
            s, zt, dinv, b.reshape(1, 32), w)

    s0 = _sc_spmm32(zt0, srcr, dstr)
    h0, zt1 = layer(32, s0, zt0, gcn_b0, gcn_w1)
    s1 = _sc_spmm32(zt1, srcr, dstr)
    h1, zt2 = layer(32, s1, zt1, gcn_b1, gcn_w2)
    s2 = _sc_spmm32(zt2, srcr, dstr)
    w3p = jnp.pad(gcn_w3, ((0, 0), (0, 15)))
    h2, zt3 = layer(16, s2, zt2, gcn_b2, w3p)
    s3 = _sc_spmm16(zt3, srcr, dstr)
    cat = _pcg(
        _h3cat_body,
        [_part(16), _rows(16), _rows(1), _full(1, 1),
         _rows(32), _rows(32), _rows(32)],
        _rows(128), jax.ShapeDtypeStruct((N, 128), f32))(
        s3, zt3, dinv, gcn_b3.reshape(1, 1), h0, h1, h2)

    keys = lax.slice(cat, (0, 96), (N, 97)).reshape(B, NPG)
    selg = _pc(_sel_body, jax.ShapeDtypeStruct((B, KP), jnp.int32))(keys)
    sel2 = selg.reshape(NW, NPG)
    xs_all = _sc_gather(cat, sel2).reshape(B * KP, 128)

    c1wr = jnp.pad(c1_w.reshape(16, 97).T, ((0, 31), (0, 0)))
    w2r = c2_w.transpose(2, 1, 0).reshape(80, 32)
    l1wr = l1_w.reshape(32, 11, 128).transpose(1, 0, 2).reshape(352, 128)
    out = _pc(_head_body, jax.ShapeDtypeStruct((B, 10), f32))(
        xs_all, c1wr, c1_b.reshape(1, 16), w2r, c2_b.reshape(1, 32),
        l1wr, l1_b.reshape(1, 128), l2_w, l2_b.reshape(1, 10))
    return out
